# baseline (speedup 1.0000x reference)
import jax
import jax.numpy as jnp
from jax import lax
from jax.experimental import pallas as pl
from jax.experimental.pallas import tpu as pltpu
from jax.experimental.pallas import tpu_sc as plsc

NC = 2
NS = 16
L = 16
NW = NC * NS

B = 16384
F = 64
B_PER_W = B // NW
C = 128
N_CHUNKS = B_PER_W // C
N_GROUPS = C // L


def _mf_kernel(uidx_hbm, iidx_hbm, uf_hbm, if_hbm, ub_hbm, ib_hbm, gb_hbm,
               out_hbm,
               uidx_v, iidx_v, urows_v, vrows_v, ub_v, ib_v,
               gb_v, out_v, acc_v, sem):
    wid = lax.axis_index("s") * NC + lax.axis_index("c")
    base = wid * B_PER_W

    pltpu.sync_copy(uidx_hbm.at[pl.ds(base, B_PER_W)], uidx_v)
    pltpu.sync_copy(iidx_hbm.at[pl.ds(base, B_PER_W)], iidx_v)
    pltpu.sync_copy(gb_hbm, gb_v.at[pl.ds(0, 1)])

    gb = gb_v[pl.ds(0, L)][0]
    lane = lax.iota(jnp.int32, L)

    for c in range(N_CHUNKS):
        coff = c * C

        def row_copies(q, make):
            x16 = uidx_v[pl.ds(coff + q * L, L)]
            y16 = iidx_v[pl.ds(coff + q * L, L)]
            for j in range(L):
                kq = q * 2 + j // 8
                kr = j % 8
                x = x16[j]
                s = jnp.right_shift(x, 3)
                i = jnp.bitwise_and(x, 7)
                y = y16[j]
                t = jnp.right_shift(y, 3)
                jj = jnp.bitwise_and(y, 7)
                make(uf_hbm.at[s, i], urows_v.at[kq, kr])
                make(if_hbm.at[t, jj], vrows_v.at[kq, kr])
                make(ub_hbm.at[s, i], ub_v.at[kq, kr])
                make(ib_hbm.at[t, jj], ib_v.at[kq, kr])

        def issue_body(q, _):
            row_copies(q, lambda a, b: pltpu.async_copy(a, b, sem))
            return 0

        lax.fori_loop(0, C // L, issue_body, 0)

        def drain_body(q, _):
            row_copies(q, lambda a, b: pltpu.make_async_copy(a, b, sem).wait())
            return 0

        lax.fori_loop(0, C // L, drain_body, 0)

        def group_body(g, _):
            for j in range(L):
                q = g * 2 + j // 8
                r = j % 8
                acc = urows_v[q, r, pl.ds(0, L)] * vrows_v[q, r, pl.ds(0, L)]
                for k in range(1, F // L):
                    acc = acc + (urows_v[q, r, pl.ds(k * L, L)]
                                 * vrows_v[q, r, pl.ds(k * L, L)])
                acc_v[j, pl.ds(0, L)] = acc
            dot = plsc.load_gather(acc_v, [lane, jnp.zeros((L,), jnp.int32)])
            for k in range(1, L):
                dot = dot + plsc.load_gather(
                    acc_v, [lane, jnp.full((L,), k, jnp.int32)])
            goff = g * L
            k16 = goff + lane
            kq16 = jnp.right_shift(k16, 3)
            kr16 = jnp.bitwise_and(k16, 7)
            zero16 = jnp.zeros((L,), jnp.int32)
            bu = plsc.load_gather(ub_v, [kq16, kr16, zero16])
            bi = plsc.load_gather(ib_v, [kq16, kr16, zero16])
            out_v[pl.ds(coff + goff, L)] = dot + bu + bi + gb
            return 0

        lax.fori_loop(0, N_GROUPS, group_body, 0)

    pltpu.sync_copy(out_v, out_hbm.at[pl.ds(base, B_PER_W)])


@jax.jit
def _run(user_idx, item_idx, uf3, if3, ub3, ib3, global_bias):
    mesh = plsc.VectorSubcoreMesh(core_axis_name="c", subcore_axis_name="s",
                                  num_cores=NC, num_subcores=NS)
    return pl.kernel(
        _mf_kernel,
        out_type=jax.ShapeDtypeStruct((B,), jnp.float32),
        mesh=mesh,
        scratch_types=[
            pltpu.VMEM((B_PER_W,), jnp.int32),
            pltpu.VMEM((B_PER_W,), jnp.int32),
            pltpu.VMEM((C // 8, 8, F), jnp.float32),
            pltpu.VMEM((C // 8, 8, F), jnp.float32),
            pltpu.VMEM((C // 8, 8, 1), jnp.float32),
            pltpu.VMEM((C // 8, 8, 1), jnp.float32),
            pltpu.VMEM((L,), jnp.float32),
            pltpu.VMEM((B_PER_W,), jnp.float32),
            pltpu.VMEM((L, L + 1), jnp.float32),
            pltpu.SemaphoreType.DMA,
        ],
        compiler_params=pltpu.CompilerParams(needs_layout_passes=False),
    )(user_idx, item_idx, uf3, if3, ub3, ib3, global_bias)


def kernel(user_idx, item_idx, user_factors, item_factors, user_bias,
           item_bias, global_bias):
    user_idx = user_idx.astype(jnp.int32)
    item_idx = item_idx.astype(jnp.int32)
    uf3 = user_factors.reshape(-1, 8, F)
    if3 = item_factors.reshape(-1, 8, F)
    ub3 = user_bias.reshape(-1, 8, 1)
    ib3 = item_bias.reshape(-1, 8, 1)
    return _run(user_idx, item_idx, uf3, if3, ub3, ib3, global_bias)


# speedup vs baseline: 1.2345x; 1.2345x over previous
import jax
import jax.numpy as jnp
from jax import lax
from jax.experimental import pallas as pl
from jax.experimental.pallas import tpu as pltpu
from jax.experimental.pallas import tpu_sc as plsc

NC = 2
NS = 16
L = 16
NW = NC * NS

B = 16384
F = 64
B_PER_W = B // NW
C = 128
N_CHUNKS = B_PER_W // C
N_GROUPS = C // L


def _mf_kernel(uidx_hbm, iidx_hbm, uf_hbm, if_hbm, ub_hbm, ib_hbm, gb_hbm,
               out_hbm,
               uidx_v, iidx_v, urows_v, vrows_v, ub_v, ib_v,
               gb_v, out_v, acc_v, sem):
    wid = lax.axis_index("s") * NC + lax.axis_index("c")
    base = wid * B_PER_W

    pltpu.sync_copy(uidx_hbm.at[pl.ds(base, B_PER_W)], uidx_v)
    pltpu.sync_copy(iidx_hbm.at[pl.ds(base, B_PER_W)], iidx_v)
    pltpu.sync_copy(gb_hbm, gb_v.at[pl.ds(0, 1)])

    gb = gb_v[pl.ds(0, L)][0]
    lane = lax.iota(jnp.int32, L)

    for c in range(N_CHUNKS):
        coff = c * C

        def row_copies(q, make):
            x16 = uidx_v[pl.ds(coff + q * L, L)]
            y16 = iidx_v[pl.ds(coff + q * L, L)]
            for j in range(L):
                kq = q * 2 + j // 8
                kr = j % 8
                x = x16[j]
                y = y16[j]
                make(uf_hbm.at[x], urows_v.at[kq, kr])
                make(if_hbm.at[y], vrows_v.at[kq, kr])
                make(ub_hbm.at[x], ub_v.at[kq, kr])
                make(ib_hbm.at[y], ib_v.at[kq, kr])

        def issue_body(q, _):
            row_copies(q, lambda a, b: pltpu.async_copy(a, b, sem))
            return 0

        lax.fori_loop(0, C // L, issue_body, 0)

        def drain_body(q, _):
            row_copies(q, lambda a, b: pltpu.make_async_copy(a, b, sem).wait())
            return 0

        lax.fori_loop(0, C // L, drain_body, 0)

        def group_body(g, _):
            for j in range(L):
                q = g * 2 + j // 8
                r = j % 8
                acc = urows_v[q, r, pl.ds(0, L)] * vrows_v[q, r, pl.ds(0, L)]
                for k in range(1, F // L):
                    acc = acc + (urows_v[q, r, pl.ds(k * L, L)]
                                 * vrows_v[q, r, pl.ds(k * L, L)])
                acc_v[j, pl.ds(0, L)] = acc
            dot = plsc.load_gather(acc_v, [lane, jnp.zeros((L,), jnp.int32)])
            for k in range(1, L):
                dot = dot + plsc.load_gather(
                    acc_v, [lane, jnp.full((L,), k, jnp.int32)])
            goff = g * L
            k16 = goff + lane
            kq16 = jnp.right_shift(k16, 3)
            kr16 = jnp.bitwise_and(k16, 7)
            zero16 = jnp.zeros((L,), jnp.int32)
            bu = plsc.load_gather(ub_v, [kq16, kr16, zero16])
            bi = plsc.load_gather(ib_v, [kq16, kr16, zero16])
            out_v[pl.ds(coff + goff, L)] = dot + bu + bi + gb
            return 0

        lax.fori_loop(0, N_GROUPS, group_body, 0)

    pltpu.sync_copy(out_v, out_hbm.at[pl.ds(base, B_PER_W)])


@jax.jit
def _run(user_idx, item_idx, uf3, if3, ub3, ib3, global_bias):
    mesh = plsc.VectorSubcoreMesh(core_axis_name="c", subcore_axis_name="s",
                                  num_cores=NC, num_subcores=NS)
    return pl.kernel(
        _mf_kernel,
        out_type=jax.ShapeDtypeStruct((B,), jnp.float32),
        mesh=mesh,
        scratch_types=[
            pltpu.VMEM((B_PER_W,), jnp.int32),
            pltpu.VMEM((B_PER_W,), jnp.int32),
            pltpu.VMEM((C // 8, 8, F), jnp.float32),
            pltpu.VMEM((C // 8, 8, F), jnp.float32),
            pltpu.VMEM((C // 8, 8, 1), jnp.float32),
            pltpu.VMEM((C // 8, 8, 1), jnp.float32),
            pltpu.VMEM((L,), jnp.float32),
            pltpu.VMEM((B_PER_W,), jnp.float32),
            pltpu.VMEM((L, L + 1), jnp.float32),
            pltpu.SemaphoreType.DMA,
        ],
        compiler_params=pltpu.CompilerParams(needs_layout_passes=False),
    )(user_idx, item_idx, uf3, if3, ub3, ib3, global_bias)


def kernel(user_idx, item_idx, user_factors, item_factors, user_bias,
           item_bias, global_bias):
    user_idx = user_idx.astype(jnp.int32)
    item_idx = item_idx.astype(jnp.int32)
    return _run(user_idx, item_idx, user_factors, item_factors, user_bias,
                item_bias, global_bias)
